# baseline (speedup 1.0000x reference)
import math

import jax
import jax.numpy as jnp
from jax.experimental import pallas as pl

_SQRT_2_OVER_PI = math.sqrt(2.0 / math.pi)


def _gelu_body(x_ref, o_ref):
    x = x_ref[...]
    inner = _SQRT_2_OVER_PI * (x + 0.044715 * (x * x * x))
    o_ref[...] = 0.5 * x * (1.0 + jnp.tanh(inner))


def kernel(x, logit_decay, log_tau, log_blend):
    del logit_decay, log_tau, log_blend
    B, T, D = x.shape
    x2 = x.reshape(B * T, D)
    bm = 512
    grid = (x2.shape[0] // bm,)
    out = pl.pallas_call(
        _gelu_body,
        grid=grid,
        in_specs=[pl.BlockSpec((bm, D), lambda i: (i, 0))],
        out_specs=pl.BlockSpec((bm, D), lambda i: (i, 0)),
        out_shape=jax.ShapeDtypeStruct(x2.shape, x2.dtype),
    )(x2)
    return out.reshape(B, T, D)


# speedup vs baseline: 1.0277x; 1.0277x over previous
import math

import jax
import jax.numpy as jnp
from jax.experimental import pallas as pl

_SQRT_2_OVER_PI = math.sqrt(2.0 / math.pi)


def _gelu_body(x_ref, o_ref):
    x = x_ref[...]
    inner = _SQRT_2_OVER_PI * (x + 0.044715 * (x * x * x))
    o_ref[...] = 0.5 * x * (1.0 + jnp.tanh(inner))


def kernel(x, logit_decay, log_tau, log_blend):
    del logit_decay, log_tau, log_blend
    B, T, D = x.shape
    x2 = x.reshape(B * T, D)
    bm = 1024
    grid = (x2.shape[0] // bm,)
    out = pl.pallas_call(
        _gelu_body,
        grid=grid,
        in_specs=[pl.BlockSpec((bm, D), lambda i: (i, 0))],
        out_specs=pl.BlockSpec((bm, D), lambda i: (i, 0)),
        out_shape=jax.ShapeDtypeStruct(x2.shape, x2.dtype),
    )(x2)
    return out.reshape(B, T, D)
